# baseline (speedup 1.0000x reference)
import functools

import jax
import jax.numpy as jnp
from jax import lax
from jax.experimental import pallas as pl
from jax.experimental.pallas import tpu as pltpu, tpu_sc as plsc

_NSVC = 10000
_NINST = 50000
_NNODE = 10000
_D = 128

_HR = 512
_JUNK = 51200

_EPAD = {100000: 114688, 150000: 163840, 75000: 81920}


def _pad_idx(a, fill):
    e = int(a.shape[0])
    ep = _EPAD[e]
    return jnp.concatenate([a, jnp.full((ep - e,), fill, jnp.int32)])


def _make_deg(e_trues):
    mesh = plsc.VectorSubcoreMesh(core_axis_name="c", subcore_axis_name="s")
    n_arr = len(e_trues)

    @functools.partial(
        pl.kernel,
        out_type=[jax.ShapeDtypeStruct((2 * _HR, 128), jnp.float32)] * n_arr,
        mesh=mesh,
        scratch_types=[pltpu.VMEM((512,), jnp.int32),
                       pltpu.VMEM((_HR, 128), jnp.float32),
                       pltpu.VMEM((128,), jnp.int32),
                       pltpu.VMEM_SHARED((_HR * n_arr, 128), jnp.float32)],
        compiler_params=pltpu.CompilerParams(needs_layout_passes=False),
    )
    def deg_kernel(*refs):
        idx_hbm = refs[:n_arr]
        zeros_hbm = refs[n_arr]
        outs = refs[n_arr + 1:2 * n_arr + 1]
        is_v, hist_v, ridx_v, sp = refs[2 * n_arr + 1:]
        c = lax.axis_index("c")
        s = lax.axis_index("s")
        wid = s * 2 + c
        ones = jnp.ones((16,), jnp.float32)
        for a, e_true in enumerate(e_trues):
            ep = _EPAD[e_true]
            et = ep // 32
            so = pl.multiple_of(s * 32, 32)
            pltpu.sync_copy(zeros_hbm.at[pl.ds(so, 32)],
                            sp.at[pl.ds(_HR * a + so, 32)])
            plsc.subcore_barrier()
            pltpu.sync_copy(zeros_hbm, hist_v)
            off0 = wid * et

            def body(sb, _, off0=off0, a=a, e_true=e_true, ref=idx_hbm[a]):
                off = pl.multiple_of(off0 + sb * 512, 512)
                pltpu.sync_copy(ref.at[pl.ds(off, 512)], is_v)
                for k in range(32):
                    v = is_v[pl.ds(16 * k, 16)]
                    pos = off + 16 * k + jnp.arange(16, dtype=jnp.int32)
                    v = jnp.where(pos < e_true, v, _JUNK)
                    r = v // 128
                    cc = v - r * 128
                    plsc.addupdate_scatter(hist_v, [r, cc], ones,
                                           mask=jnp.ones((16,), jnp.bool_))
                return 0

            lax.fori_loop(0, et // 512, body, 0)
            for q in range(4):
                for j in range(8):
                    ridx_v[pl.ds(16 * j, 16)] = (
                        jnp.arange(16, dtype=jnp.int32)
                        + 16 * j + 128 * q + _HR * a)
                pltpu.sync_copy(hist_v.at[pl.ds(128 * q, 128)],
                                sp.at[ridx_v], add=True)
        plsc.subcore_barrier()

        @pl.when(s == 0)
        def _():
            for a in range(n_arr):
                pltpu.sync_copy(sp.at[pl.ds(_HR * a, _HR)],
                                outs[a].at[pl.ds(pl.multiple_of(c * _HR, _HR),
                                                 _HR)])

    return deg_kernel


def _make_agg(n_src, n_dst, ch, n_chunks, e_true):
    ep = _EPAD[e_true]
    et = ep // 16
    nsb = et // 512
    chd = ch // 16
    mesh = plsc.VectorSubcoreMesh(core_axis_name="c", subcore_axis_name="s")

    @functools.partial(
        pl.kernel,
        out_type=jax.ShapeDtypeStruct((n_chunks * ch, _D), jnp.float32),
        mesh=mesh,
        scratch_types=[pltpu.VMEM((512,), jnp.int32),
                       pltpu.VMEM((512,), jnp.int32),
                       pltpu.VMEM((4, 128), jnp.int32),
                       pltpu.VMEM((128, _D), jnp.float32),
                       pltpu.VMEM((128, _D), jnp.float32),
                       pltpu.VMEM_SHARED((ch + 16, _D), jnp.float32),
                       pltpu.SemaphoreType.DMA,
                       pltpu.SemaphoreType.DMA],
        compiler_params=pltpu.CompilerParams(needs_layout_passes=False),
    )
    def agg_kernel(xn_hbm, srcp_hbm, dstp_hbm, zeros_hbm, a_out,
                   gs_v, ds_v, lidx_v, row0_v, row1_v, sp, semA, semB):
        c = lax.axis_index("c")
        s = lax.axis_index("s")
        for slot in range(n_chunks // 2):
            ck = 2 * slot + c
            base = ck * ch
            zo = pl.multiple_of(s * chd, chd)
            pltpu.sync_copy(zeros_hbm.at[pl.ds(0, chd)], sp.at[pl.ds(zo, chd)])

            @pl.when(s == 0)
            def _():
                pltpu.sync_copy(zeros_hbm.at[pl.ds(0, 16)],
                                sp.at[pl.ds(ch, 16)])

            plsc.subcore_barrier()

            def body(sb, _, base=base):
                off = pl.multiple_of(s * et + sb * 512, 512)
                pltpu.sync_copy(srcp_hbm.at[pl.ds(off, 512)], gs_v)
                pltpu.sync_copy(dstp_hbm.at[pl.ds(off, 512)], ds_v)
                for k in range(32):
                    j, i = divmod(k, 8)
                    d = ds_v[pl.ds(16 * k, 16)]
                    lx = d - base
                    m = jnp.logical_and(lx >= 0, lx < ch)
                    lidx_v[j, pl.ds(16 * i, 16)] = jnp.where(m, lx, ch)

                def fire(j, buf, sem):
                    return pltpu.async_copy(
                        xn_hbm.at[gs_v.at[pl.ds(128 * j, 128)]], buf, sem)

                cp0 = fire(0, row0_v, semA)
                cp1 = fire(1, row1_v, semB)
                cp0.wait()
                pltpu.sync_copy(row0_v, sp.at[lidx_v.at[0]], add=True)
                cp2 = fire(2, row0_v, semA)
                cp1.wait()
                pltpu.sync_copy(row1_v, sp.at[lidx_v.at[1]], add=True)
                cp3 = fire(3, row1_v, semB)
                cp2.wait()
                pltpu.sync_copy(row0_v, sp.at[lidx_v.at[2]], add=True)
                cp3.wait()
                pltpu.sync_copy(row1_v, sp.at[lidx_v.at[3]], add=True)
                return 0

            lax.fori_loop(0, nsb, body, 0)
            plsc.subcore_barrier()
            pltpu.sync_copy(sp.at[pl.ds(pl.multiple_of(s * chd, chd), chd)],
                            a_out.at[pl.ds(pl.multiple_of(
                                ck * ch + s * chd, chd), chd)])
            plsc.subcore_barrier()

    return agg_kernel


def _combine(n_out, parts):
    k_rel = len(parts)
    nb = n_out // 80

    def body(*refs):
        out_ref = refs[-1]
        acc = jnp.zeros((80, _D), jnp.float32)
        for r in range(k_rel):
            a = refs[4 * r][...]
            w = refs[4 * r + 1][...]
            rs = refs[4 * r + 2][...]
            b = refs[4 * r + 3][...]
            acc = acc + jnp.dot(a, w,
                                preferred_element_type=jnp.float32) * rs + b
        out_ref[...] = jnp.maximum(acc * (1.0 / k_rel), 0.0)

    in_specs = []
    args = []
    for (a, w, rs, b) in parts:
        in_specs += [
            pl.BlockSpec((80, _D), lambda i: (i, 0)),
            pl.BlockSpec((_D, _D), lambda i: (0, 0)),
            pl.BlockSpec((80, 1), lambda i: (i, 0)),
            pl.BlockSpec((1, _D), lambda i: (0, 0)),
        ]
        args += [a, w, jnp.reshape(rs, (-1, 1)), jnp.reshape(b, (1, _D))]
    return pl.pallas_call(
        body,
        grid=(nb,),
        in_specs=in_specs,
        out_specs=pl.BlockSpec((80, _D), lambda i: (i, 0)),
        out_shape=jax.ShapeDtypeStruct((n_out, _D), jnp.float32),
    )(*args)


def kernel(x_svc, x_instance, x_node,
           svc_call_src, svc_call_dst, inst_node_src, inst_node_dst,
           node_inst_src, node_inst_dst, inst_inst_src, inst_inst_dst,
           svc_inst_src, svc_inst_dst, inst_svc_src, inst_svc_dst,
           W_svc_call, b_svc_call, W_inst_node, b_inst_node,
           W_node_inst, b_node_inst, W_inst_inst, b_inst_inst,
           W_svc_inst, b_svc_inst, W_inst_svc, b_inst_svc):
    rels = [
        ("svc_call", svc_call_src, svc_call_dst, x_svc,
         W_svc_call, b_svc_call, _NSVC, _NSVC),
        ("inst_node", inst_node_src, inst_node_dst, x_instance,
         W_inst_node, b_inst_node, _NINST, _NNODE),
        ("node_inst", node_inst_src, node_inst_dst, x_node,
         W_node_inst, b_node_inst, _NNODE, _NINST),
        ("inst_inst", inst_inst_src, inst_inst_dst, x_instance,
         W_inst_inst, b_inst_inst, _NINST, _NINST),
        ("svc_inst", svc_inst_src, svc_inst_dst, x_svc,
         W_svc_inst, b_svc_inst, _NSVC, _NINST),
        ("inst_svc", inst_svc_src, inst_svc_dst, x_instance,
         W_inst_svc, b_inst_svc, _NINST, _NSVC),
    ]
    srcs_p = [_pad_idx(r[1], 0) for r in rels]
    dsts_p = [_pad_idx(r[2], -1) for r in rels]
    e_trues = [int(r[1].shape[0]) for r in rels]

    zeros_h = jnp.zeros((_HR, 128), jnp.float32)
    deg_outs = _make_deg(e_trues + e_trues)(*(srcs_p + dsts_p), zeros_h)

    def fin_deg(o, n):
        return jnp.maximum((o[:_HR] + o[_HR:]).reshape(-1)[:n], 1.0)

    deg_src = [fin_deg(deg_outs[i], rels[i][6]) for i in range(6)]
    deg_dst = [fin_deg(deg_outs[6 + i], rels[i][7]) for i in range(6)]

    aggs = []
    for i, r in enumerate(rels):
        n_src, n_dst = r[6], r[7]
        ch = 8448 if n_dst == _NINST else 5120
        n_chunks = 6 if n_dst == _NINST else 2
        xn = r[3] * lax.rsqrt(deg_src[i])[:, None]
        zeros_c = jnp.zeros((ch + 16, _D), jnp.float32)
        a = _make_agg(n_src, n_dst, ch, n_chunks, e_trues[i])(
            xn, srcs_p[i], dsts_p[i], zeros_c)
        aggs.append(a)

    rs_dst = [lax.rsqrt(deg_dst[i]) for i in range(6)]

    node_feat = _combine(_NNODE, [(aggs[1], rels[1][4], rs_dst[1], rels[1][5])])
    inst_feat = _combine(_NINST, [
        (aggs[2], rels[2][4], rs_dst[2], rels[2][5]),
        (aggs[3], rels[3][4], rs_dst[3], rels[3][5]),
        (aggs[4], rels[4][4], rs_dst[4], rels[4][5]),
    ])
    svc_feat = _combine(_NSVC, [
        (aggs[0], rels[0][4], rs_dst[0], rels[0][5]),
        (aggs[5], rels[5][4], rs_dst[5], rels[5][5]),
    ])
    return jnp.concatenate([node_feat, inst_feat, svc_feat], 0)


# speedup vs baseline: 5.6570x; 5.6570x over previous
import functools

import jax
import jax.numpy as jnp
from jax import lax
from jax.experimental import pallas as pl
from jax.experimental.pallas import tpu as pltpu, tpu_sc as plsc

_NSVC = 10000
_NINST = 50000
_NNODE = 10000
_D = 128

_HR = 512
_JUNK = 51200

_EPAD = {100000: 114688, 150000: 163840, 75000: 81920}


def _pad_idx(a, fill):
    e = int(a.shape[0])
    ep = _EPAD[e]
    return jnp.concatenate([a, jnp.full((ep - e,), fill, jnp.int32)])


def _make_deg(e_trues):
    mesh = plsc.VectorSubcoreMesh(core_axis_name="c", subcore_axis_name="s")
    n_arr = len(e_trues)

    @functools.partial(
        pl.kernel,
        out_type=[jax.ShapeDtypeStruct((2 * _HR, 128), jnp.float32)] * n_arr,
        mesh=mesh,
        scratch_types=[pltpu.VMEM((512,), jnp.int32),
                       pltpu.VMEM((_HR, 128), jnp.float32),
                       pltpu.VMEM((128,), jnp.int32),
                       pltpu.VMEM_SHARED((_HR * n_arr, 128), jnp.float32)],
        compiler_params=pltpu.CompilerParams(needs_layout_passes=False),
    )
    def deg_kernel(*refs):
        idx_hbm = refs[:n_arr]
        zeros_hbm = refs[n_arr]
        outs = refs[n_arr + 1:2 * n_arr + 1]
        is_v, hist_v, ridx_v, sp = refs[2 * n_arr + 1:]
        c = lax.axis_index("c")
        s = lax.axis_index("s")
        wid = s * 2 + c
        ones = jnp.ones((16,), jnp.float32)
        for a, e_true in enumerate(e_trues):
            ep = _EPAD[e_true]
            et = ep // 32
            so = pl.multiple_of(s * 32, 32)
            pltpu.sync_copy(zeros_hbm.at[pl.ds(so, 32)],
                            sp.at[pl.ds(_HR * a + so, 32)])
            plsc.subcore_barrier()
            pltpu.sync_copy(zeros_hbm, hist_v)
            off0 = wid * et

            def body(sb, _, off0=off0, a=a, e_true=e_true, ref=idx_hbm[a]):
                off = pl.multiple_of(off0 + sb * 512, 512)
                pltpu.sync_copy(ref.at[pl.ds(off, 512)], is_v)
                for k in range(32):
                    v = is_v[pl.ds(16 * k, 16)]
                    pos = off + 16 * k + jnp.arange(16, dtype=jnp.int32)
                    v = jnp.where(pos < e_true, v, _JUNK)
                    r = v // 128
                    cc = v - r * 128
                    plsc.addupdate_scatter(hist_v, [r, cc], ones,
                                           mask=jnp.ones((16,), jnp.bool_))
                return 0

            lax.fori_loop(0, et // 512, body, 0)
            for q in range(4):
                for j in range(8):
                    ridx_v[pl.ds(16 * j, 16)] = (
                        jnp.arange(16, dtype=jnp.int32)
                        + 16 * j + 128 * q + _HR * a)
                pltpu.sync_copy(hist_v.at[pl.ds(128 * q, 128)],
                                sp.at[ridx_v], add=True)
        plsc.subcore_barrier()

        @pl.when(s == 0)
        def _():
            for a in range(n_arr):
                pltpu.sync_copy(sp.at[pl.ds(_HR * a, _HR)],
                                outs[a].at[pl.ds(pl.multiple_of(c * _HR, _HR),
                                                 _HR)])

    return deg_kernel


def _make_agg(n_src, n_dst, ch, n_chunks, e_true):
    ep = _EPAD[e_true]
    et = ep // 16
    nsb = et // 512
    chd = ch // 16
    mesh = plsc.VectorSubcoreMesh(core_axis_name="c", subcore_axis_name="s")

    @functools.partial(
        pl.kernel,
        out_type=jax.ShapeDtypeStruct((n_chunks * ch, _D), jnp.float32),
        mesh=mesh,
        scratch_types=[pltpu.VMEM((512,), jnp.int32),
                       pltpu.VMEM((512,), jnp.int32),
                       pltpu.VMEM((4, 128), jnp.int32),
                       pltpu.VMEM((128, _D), jnp.float32),
                       pltpu.VMEM_SHARED((ch + 16, _D), jnp.float32),
                       pltpu.SemaphoreType.DMA],
        compiler_params=pltpu.CompilerParams(needs_layout_passes=False),
    )
    def agg_kernel(xn_hbm, srcp_hbm, dstp_hbm, zeros_hbm, a_out,
                   gs_v, ds_v, lidx_v, rows_v, sp, sem):
        c = lax.axis_index("c")
        s = lax.axis_index("s")
        for slot in range(n_chunks // 2):
            ck = 2 * slot + c
            base = ck * ch
            zo = pl.multiple_of(s * chd, chd)
            pltpu.sync_copy(zeros_hbm.at[pl.ds(0, chd)], sp.at[pl.ds(zo, chd)])

            @pl.when(s == 0)
            def _():
                pltpu.sync_copy(zeros_hbm.at[pl.ds(0, 16)],
                                sp.at[pl.ds(ch, 16)])

            plsc.subcore_barrier()

            def body(sb, _, base=base):
                off = pl.multiple_of(s * et + sb * 512, 512)

                @pl.when(off < e_true)
                def _():
                    pltpu.sync_copy(srcp_hbm.at[pl.ds(off, 512)], gs_v)
                    pltpu.sync_copy(dstp_hbm.at[pl.ds(off, 512)], ds_v)
                    for k in range(32):
                        j, i = divmod(k, 8)
                        d = ds_v[pl.ds(16 * k, 16)]
                        lx = d - base
                        m = jnp.logical_and(lx >= 0, lx < ch)
                        lidx_v[j, pl.ds(16 * i, 16)] = jnp.where(m, lx, ch)
                    for j in range(4):
                        pltpu.async_copy(
                            xn_hbm.at[gs_v.at[pl.ds(128 * j, 128)]],
                            rows_v, sem).wait()
                        pltpu.sync_copy(rows_v, sp.at[lidx_v.at[j]],
                                        add=True)

                return 0

            lax.fori_loop(0, nsb, body, 0)
            plsc.subcore_barrier()
            pltpu.sync_copy(sp.at[pl.ds(pl.multiple_of(s * chd, chd), chd)],
                            a_out.at[pl.ds(pl.multiple_of(
                                ck * ch + s * chd, chd), chd)])
            plsc.subcore_barrier()

    return agg_kernel


def _combine(n_out, parts):
    k_rel = len(parts)
    nb = n_out // 80

    def body(*refs):
        out_ref = refs[-1]
        acc = jnp.zeros((80, _D), jnp.float32)
        for r in range(k_rel):
            a = refs[4 * r][...]
            w = refs[4 * r + 1][...]
            rs = refs[4 * r + 2][...]
            b = refs[4 * r + 3][...]
            acc = acc + jnp.dot(a, w,
                                preferred_element_type=jnp.float32) * rs + b
        out_ref[...] = jnp.maximum(acc * (1.0 / k_rel), 0.0)

    in_specs = []
    args = []
    for (a, w, rs, b) in parts:
        in_specs += [
            pl.BlockSpec((80, _D), lambda i: (i, 0)),
            pl.BlockSpec((_D, _D), lambda i: (0, 0)),
            pl.BlockSpec((80, 1), lambda i: (i, 0)),
            pl.BlockSpec((1, _D), lambda i: (0, 0)),
        ]
        args += [a, w, jnp.reshape(rs, (-1, 1)), jnp.reshape(b, (1, _D))]
    return pl.pallas_call(
        body,
        grid=(nb,),
        in_specs=in_specs,
        out_specs=pl.BlockSpec((80, _D), lambda i: (i, 0)),
        out_shape=jax.ShapeDtypeStruct((n_out, _D), jnp.float32),
    )(*args)


def kernel(x_svc, x_instance, x_node,
           svc_call_src, svc_call_dst, inst_node_src, inst_node_dst,
           node_inst_src, node_inst_dst, inst_inst_src, inst_inst_dst,
           svc_inst_src, svc_inst_dst, inst_svc_src, inst_svc_dst,
           W_svc_call, b_svc_call, W_inst_node, b_inst_node,
           W_node_inst, b_node_inst, W_inst_inst, b_inst_inst,
           W_svc_inst, b_svc_inst, W_inst_svc, b_inst_svc):
    rels = [
        ("svc_call", svc_call_src, svc_call_dst, x_svc,
         W_svc_call, b_svc_call, _NSVC, _NSVC),
        ("inst_node", inst_node_src, inst_node_dst, x_instance,
         W_inst_node, b_inst_node, _NINST, _NNODE),
        ("node_inst", node_inst_src, node_inst_dst, x_node,
         W_node_inst, b_node_inst, _NNODE, _NINST),
        ("inst_inst", inst_inst_src, inst_inst_dst, x_instance,
         W_inst_inst, b_inst_inst, _NINST, _NINST),
        ("svc_inst", svc_inst_src, svc_inst_dst, x_svc,
         W_svc_inst, b_svc_inst, _NSVC, _NINST),
        ("inst_svc", inst_svc_src, inst_svc_dst, x_instance,
         W_inst_svc, b_inst_svc, _NINST, _NSVC),
    ]
    srcs_p = [_pad_idx(r[1], 0) for r in rels]
    dsts_p = [_pad_idx(r[2], -1) for r in rels]
    e_trues = [int(r[1].shape[0]) for r in rels]

    zeros_h = jnp.zeros((_HR, 128), jnp.float32)
    deg_outs = _make_deg(e_trues + e_trues)(*(srcs_p + dsts_p), zeros_h)

    def fin_deg(o, n):
        return jnp.maximum((o[:_HR] + o[_HR:]).reshape(-1)[:n], 1.0)

    deg_src = [fin_deg(deg_outs[i], rels[i][6]) for i in range(6)]
    deg_dst = [fin_deg(deg_outs[6 + i], rels[i][7]) for i in range(6)]

    aggs = []
    for i, r in enumerate(rels):
        n_src, n_dst = r[6], r[7]
        ch = 12800 if n_dst == _NINST else 5120
        n_chunks = 4 if n_dst == _NINST else 2
        xn = r[3] * lax.rsqrt(deg_src[i])[:, None]
        zeros_c = jnp.zeros((ch + 16, _D), jnp.float32)
        a = _make_agg(n_src, n_dst, ch, n_chunks, e_trues[i])(
            xn, srcs_p[i], dsts_p[i], zeros_c)
        aggs.append(a)

    rs_dst = [lax.rsqrt(deg_dst[i]) for i in range(6)]

    node_feat = _combine(_NNODE, [(aggs[1], rels[1][4], rs_dst[1], rels[1][5])])
    inst_feat = _combine(_NINST, [
        (aggs[2], rels[2][4], rs_dst[2], rels[2][5]),
        (aggs[3], rels[3][4], rs_dst[3], rels[3][5]),
        (aggs[4], rels[4][4], rs_dst[4], rels[4][5]),
    ])
    svc_feat = _combine(_NSVC, [
        (aggs[0], rels[0][4], rs_dst[0], rels[0][5]),
        (aggs[5], rels[5][4], rs_dst[5], rels[5][5]),
    ])
    return jnp.concatenate([node_feat, inst_feat, svc_feat], 0)


# speedup vs baseline: 5.7310x; 1.0131x over previous
import functools

import jax
import jax.numpy as jnp
from jax import lax
from jax.experimental import pallas as pl
from jax.experimental.pallas import tpu as pltpu, tpu_sc as plsc

_NSVC = 10000
_NINST = 50000
_NNODE = 10000
_D = 128

_HR = 512
_JUNK = 51200

_EPAD = {100000: 114688, 150000: 163840, 75000: 81920}


def _pad_idx(a, fill):
    e = int(a.shape[0])
    ep = _EPAD[e]
    return jnp.concatenate([a, jnp.full((ep - e,), fill, jnp.int32)])


def _make_deg(e_trues):
    mesh = plsc.VectorSubcoreMesh(core_axis_name="c", subcore_axis_name="s")
    n_arr = len(e_trues)

    @functools.partial(
        pl.kernel,
        out_type=[jax.ShapeDtypeStruct((2 * _HR, 128), jnp.float32)] * n_arr,
        mesh=mesh,
        scratch_types=[pltpu.VMEM((512,), jnp.int32),
                       pltpu.VMEM((_HR, 128), jnp.float32),
                       pltpu.VMEM((128,), jnp.int32),
                       pltpu.VMEM_SHARED((_HR * n_arr, 128), jnp.float32)],
        compiler_params=pltpu.CompilerParams(needs_layout_passes=False),
    )
    def deg_kernel(*refs):
        idx_hbm = refs[:n_arr]
        zeros_hbm = refs[n_arr]
        outs = refs[n_arr + 1:2 * n_arr + 1]
        is_v, hist_v, ridx_v, sp = refs[2 * n_arr + 1:]
        c = lax.axis_index("c")
        s = lax.axis_index("s")
        wid = s * 2 + c
        ones = jnp.ones((16,), jnp.float32)
        for a, e_true in enumerate(e_trues):
            ep = _EPAD[e_true]
            et = ep // 32
            so = pl.multiple_of(s * 32, 32)
            pltpu.sync_copy(zeros_hbm.at[pl.ds(so, 32)],
                            sp.at[pl.ds(_HR * a + so, 32)])
            plsc.subcore_barrier()
            pltpu.sync_copy(zeros_hbm, hist_v)
            off0 = wid * et

            def body(sb, _, off0=off0, a=a, e_true=e_true, ref=idx_hbm[a]):
                off = pl.multiple_of(off0 + sb * 512, 512)
                pltpu.sync_copy(ref.at[pl.ds(off, 512)], is_v)
                for k in range(32):
                    v = is_v[pl.ds(16 * k, 16)]
                    pos = off + 16 * k + jnp.arange(16, dtype=jnp.int32)
                    v = jnp.where(pos < e_true, v, _JUNK)
                    r = v // 128
                    cc = v - r * 128
                    plsc.addupdate_scatter(hist_v, [r, cc], ones,
                                           mask=jnp.ones((16,), jnp.bool_))
                return 0

            lax.fori_loop(0, et // 512, body, 0)
            for q in range(4):
                for j in range(8):
                    ridx_v[pl.ds(16 * j, 16)] = (
                        jnp.arange(16, dtype=jnp.int32)
                        + 16 * j + 128 * q + _HR * a)
                pltpu.sync_copy(hist_v.at[pl.ds(128 * q, 128)],
                                sp.at[ridx_v], add=True)
        plsc.subcore_barrier()

        @pl.when(s == 0)
        def _():
            for a in range(n_arr):
                pltpu.sync_copy(sp.at[pl.ds(_HR * a, _HR)],
                                outs[a].at[pl.ds(pl.multiple_of(c * _HR, _HR),
                                                 _HR)])

    return deg_kernel


def _make_agg(n_src, n_dst, ch, n_chunks, e_true):
    ep = _EPAD[e_true]
    et = ep // 16
    nsb = et // 512
    chd = ch // 16
    mesh = plsc.VectorSubcoreMesh(core_axis_name="c", subcore_axis_name="s")

    @functools.partial(
        pl.kernel,
        out_type=jax.ShapeDtypeStruct((n_chunks * ch, _D), jnp.float32),
        mesh=mesh,
        scratch_types=[pltpu.VMEM((512,), jnp.int32),
                       pltpu.VMEM((512,), jnp.int32),
                       pltpu.VMEM((4, 128), jnp.int32),
                       pltpu.VMEM((128, _D), jnp.float32),
                       pltpu.VMEM_SHARED((ch + 16, _D), jnp.float32),
                       pltpu.SemaphoreType.DMA],
        compiler_params=pltpu.CompilerParams(needs_layout_passes=False),
    )
    def agg_kernel(xn_hbm, srcp_hbm, dstp_hbm, zeros_hbm, a_out,
                   gs_v, ds_v, lidx_v, rows_v, sp, sem):
        c = lax.axis_index("c")
        s = lax.axis_index("s")
        for slot in range(n_chunks // 2):
            ck = 2 * slot + c
            base = ck * ch
            zo = pl.multiple_of(s * chd, chd)
            pltpu.sync_copy(zeros_hbm.at[pl.ds(0, chd)], sp.at[pl.ds(zo, chd)])

            @pl.when(s == 0)
            def _():
                pltpu.sync_copy(zeros_hbm.at[pl.ds(0, 16)],
                                sp.at[pl.ds(ch, 16)])

            plsc.subcore_barrier()

            def body(sb, _, base=base):
                off = pl.multiple_of(s * et + sb * 512, 512)

                @pl.when(off < e_true)
                def _():
                    pltpu.sync_copy(srcp_hbm.at[pl.ds(off, 512)], gs_v)
                    pltpu.sync_copy(dstp_hbm.at[pl.ds(off, 512)], ds_v)
                    for k in range(32):
                        j, i = divmod(k, 8)
                        d = ds_v[pl.ds(16 * k, 16)]
                        lx = d - base
                        m = jnp.logical_and(lx >= 0, lx < ch)
                        lidx_v[j, pl.ds(16 * i, 16)] = jnp.where(
                            m, lx, ch + s)
                    for j in range(4):
                        pltpu.async_copy(
                            xn_hbm.at[gs_v.at[pl.ds(128 * j, 128)]],
                            rows_v, sem).wait()
                        pltpu.sync_copy(rows_v, sp.at[lidx_v.at[j]],
                                        add=True)

                return 0

            lax.fori_loop(0, nsb, body, 0)
            plsc.subcore_barrier()
            pltpu.sync_copy(sp.at[pl.ds(pl.multiple_of(s * chd, chd), chd)],
                            a_out.at[pl.ds(pl.multiple_of(
                                ck * ch + s * chd, chd), chd)])
            plsc.subcore_barrier()

    return agg_kernel


def _combine(n_out, parts):
    k_rel = len(parts)
    nb = n_out // 80

    def body(*refs):
        out_ref = refs[-1]
        acc = jnp.zeros((80, _D), jnp.float32)
        for r in range(k_rel):
            a = refs[4 * r][...]
            w = refs[4 * r + 1][...]
            rs = refs[4 * r + 2][...]
            b = refs[4 * r + 3][...]
            acc = acc + jnp.dot(a, w,
                                preferred_element_type=jnp.float32) * rs + b
        out_ref[...] = jnp.maximum(acc * (1.0 / k_rel), 0.0)

    in_specs = []
    args = []
    for (a, w, rs, b) in parts:
        in_specs += [
            pl.BlockSpec((80, _D), lambda i: (i, 0)),
            pl.BlockSpec((_D, _D), lambda i: (0, 0)),
            pl.BlockSpec((80, 1), lambda i: (i, 0)),
            pl.BlockSpec((1, _D), lambda i: (0, 0)),
        ]
        args += [a, w, jnp.reshape(rs, (-1, 1)), jnp.reshape(b, (1, _D))]
    return pl.pallas_call(
        body,
        grid=(nb,),
        in_specs=in_specs,
        out_specs=pl.BlockSpec((80, _D), lambda i: (i, 0)),
        out_shape=jax.ShapeDtypeStruct((n_out, _D), jnp.float32),
    )(*args)


def kernel(x_svc, x_instance, x_node,
           svc_call_src, svc_call_dst, inst_node_src, inst_node_dst,
           node_inst_src, node_inst_dst, inst_inst_src, inst_inst_dst,
           svc_inst_src, svc_inst_dst, inst_svc_src, inst_svc_dst,
           W_svc_call, b_svc_call, W_inst_node, b_inst_node,
           W_node_inst, b_node_inst, W_inst_inst, b_inst_inst,
           W_svc_inst, b_svc_inst, W_inst_svc, b_inst_svc):
    rels = [
        ("svc_call", svc_call_src, svc_call_dst, x_svc,
         W_svc_call, b_svc_call, _NSVC, _NSVC),
        ("inst_node", inst_node_src, inst_node_dst, x_instance,
         W_inst_node, b_inst_node, _NINST, _NNODE),
        ("node_inst", node_inst_src, node_inst_dst, x_node,
         W_node_inst, b_node_inst, _NNODE, _NINST),
        ("inst_inst", inst_inst_src, inst_inst_dst, x_instance,
         W_inst_inst, b_inst_inst, _NINST, _NINST),
        ("svc_inst", svc_inst_src, svc_inst_dst, x_svc,
         W_svc_inst, b_svc_inst, _NSVC, _NINST),
        ("inst_svc", inst_svc_src, inst_svc_dst, x_instance,
         W_inst_svc, b_inst_svc, _NINST, _NSVC),
    ]
    srcs_p = [_pad_idx(r[1], 0) for r in rels]
    dsts_p = [_pad_idx(r[2], -1) for r in rels]
    e_trues = [int(r[1].shape[0]) for r in rels]

    zeros_h = jnp.zeros((_HR, 128), jnp.float32)
    deg_outs = _make_deg(e_trues + e_trues)(*(srcs_p + dsts_p), zeros_h)

    def fin_deg(o, n):
        return jnp.maximum((o[:_HR] + o[_HR:]).reshape(-1)[:n], 1.0)

    deg_src = [fin_deg(deg_outs[i], rels[i][6]) for i in range(6)]
    deg_dst = [fin_deg(deg_outs[6 + i], rels[i][7]) for i in range(6)]

    aggs = []
    for i, r in enumerate(rels):
        n_src, n_dst = r[6], r[7]
        ch = 12800 if n_dst == _NINST else 5120
        n_chunks = 4 if n_dst == _NINST else 2
        xn = r[3] * lax.rsqrt(deg_src[i])[:, None]
        zeros_c = jnp.zeros((ch + 16, _D), jnp.float32)
        a = _make_agg(n_src, n_dst, ch, n_chunks, e_trues[i])(
            xn, srcs_p[i], dsts_p[i], zeros_c)
        aggs.append(a)

    rs_dst = [lax.rsqrt(deg_dst[i]) for i in range(6)]

    node_feat = _combine(_NNODE, [(aggs[1], rels[1][4], rs_dst[1], rels[1][5])])
    inst_feat = _combine(_NINST, [
        (aggs[2], rels[2][4], rs_dst[2], rels[2][5]),
        (aggs[3], rels[3][4], rs_dst[3], rels[3][5]),
        (aggs[4], rels[4][4], rs_dst[4], rels[4][5]),
    ])
    svc_feat = _combine(_NSVC, [
        (aggs[0], rels[0][4], rs_dst[0], rels[0][5]),
        (aggs[5], rels[5][4], rs_dst[5], rels[5][5]),
    ])
    return jnp.concatenate([node_feat, inst_feat, svc_feat], 0)
